# Initial kernel scaffold; baseline (speedup 1.0000x reference)
import functools

import jax
import jax.numpy as jnp
from jax.experimental import pallas as pl

_F32 = jnp.float32
_BN_EPS = 1e-5
_BIG = 1e30


def _finalize_stats(st_ref, g_ref, be_ref, cnt):
    s = st_ref[0, :]
    ss = st_ref[1, :]
    mu = s / cnt
    var = ss / cnt - mu * mu
    sc = g_ref[0, :] * jax.lax.rsqrt(var + _BN_EPS)
    sh = be_ref[0, :] - mu * sc
    return sc, sh


def _accum_stats(st_ref, x, first):
    ps = jnp.sum(x, axis=0)
    pss = jnp.sum(x * x, axis=0)
    part = jnp.concatenate([ps[None, :], pss[None, :]], axis=0)
    prev = st_ref[...]
    st_ref[...] = jnp.where(first, part, prev + part)


def _interp_body(refs, *, M, cnt_prev, has_p1, norm_p2):
    i = 0
    xyz1_ref = refs[i]; i += 1
    xyz2_ref = refs[i]; i += 1
    p2_ref = refs[i]; i += 1
    if norm_p2:
        sp_ref = refs[i]; i += 1
        gp_ref = refs[i]; i += 1
        bep_ref = refs[i]; i += 1
    if has_p1:
        p1_ref = refs[i]; i += 1
        w0a_ref = refs[i]; i += 1
    w0b_ref = refs[i]; i += 1
    b0_ref = refs[i]; i += 1
    x_ref = refs[i]; i += 1
    st_ref = refs[i]; i += 1

    b = pl.program_id(0)
    t = pl.program_id(1)
    first = jnp.logical_and(b == 0, t == 0)

    q = xyz1_ref[0]
    k = xyz2_ref[0]
    p2 = p2_ref[0]
    if norm_p2:
        sc, sh = _finalize_stats(sp_ref, gp_ref, bep_ref, cnt_prev)
        p2 = jnp.maximum(p2 * sc[None, :] + sh[None, :], 0.0)

    qq = jnp.sum(q * q, axis=1, keepdims=True)
    kk = jnp.sum(k * k, axis=1)[None, :]
    qk = jax.lax.dot_general(q, k, (((1,), (1,)), ((), ())),
                             preferred_element_type=_F32)
    d = qq - 2.0 * qk + kk

    iota = jax.lax.broadcasted_iota(jnp.int32, d.shape, 1)
    dd = d
    recs = []
    sels = []
    for _ in range(3):
        mv = jnp.min(dd, axis=1, keepdims=True)
        idx = jnp.min(jnp.where(dd == mv, iota, M), axis=1, keepdims=True)
        sel = iota == idx
        dd = jnp.where(sel, _BIG, dd)
        recs.append(1.0 / (mv + 1e-8))
        sels.append(sel)
    wsum = recs[0] + recs[1] + recs[2]
    wmat = ((recs[0] / wsum) * sels[0].astype(_F32)
            + (recs[1] / wsum) * sels[1].astype(_F32)
            + (recs[2] / wsum) * sels[2].astype(_F32))

    interp = jnp.dot(wmat, p2, preferred_element_type=_F32)
    x = jnp.dot(interp, w0b_ref[...], preferred_element_type=_F32)
    if has_p1:
        x = x + jnp.dot(p1_ref[0], w0a_ref[...], preferred_element_type=_F32)
    x = x + b0_ref[0, :][None, :]
    x_ref[0] = x
    _accum_stats(st_ref, x, first)


def _level_pass_a(xyz1, xyz2, p2, st_prev, g_prev, be_prev, p1, w0, b0, qt):
    B, N, _ = xyz1.shape
    M = xyz2.shape[1]
    C2 = p2.shape[2]
    C0 = w0.shape[0]
    norm_p2 = st_prev is not None
    has_p1 = p1 is not None
    cnt_prev = float(B * M)

    inputs = [xyz1, xyz2, p2]
    specs = [
        pl.BlockSpec((1, qt, 3), lambda b, t: (b, t, 0)),
        pl.BlockSpec((1, M, 3), lambda b, t: (b, 0, 0)),
        pl.BlockSpec((1, M, C2), lambda b, t: (b, 0, 0)),
    ]
    if norm_p2:
        inputs += [st_prev, g_prev.reshape(1, -1), be_prev.reshape(1, -1)]
        specs += [
            pl.BlockSpec((2, C2), lambda b, t: (0, 0)),
            pl.BlockSpec((1, C2), lambda b, t: (0, 0)),
            pl.BlockSpec((1, C2), lambda b, t: (0, 0)),
        ]
    if has_p1:
        C1 = p1.shape[2]
        w0a = jnp.transpose(w0[:, :C1])
        w0b = jnp.transpose(w0[:, C1:])
        inputs += [p1, w0a]
        specs += [
            pl.BlockSpec((1, qt, C1), lambda b, t: (b, t, 0)),
            pl.BlockSpec((C1, C0), lambda b, t: (0, 0)),
        ]
    else:
        w0b = jnp.transpose(w0)
    inputs += [w0b, b0.reshape(1, -1)]
    specs += [
        pl.BlockSpec((C2, C0), lambda b, t: (0, 0)),
        pl.BlockSpec((1, C0), lambda b, t: (0, 0)),
    ]

    body = functools.partial(
        _interp_body, M=M, cnt_prev=cnt_prev, has_p1=has_p1, norm_p2=norm_p2)

    def wrapped(*refs):
        body(refs)

    x, st = pl.pallas_call(
        wrapped,
        grid=(B, N // qt),
        in_specs=specs,
        out_specs=[
            pl.BlockSpec((1, qt, C0), lambda b, t: (b, t, 0)),
            pl.BlockSpec((2, C0), lambda b, t: (0, 0)),
        ],
        out_shape=[
            jax.ShapeDtypeStruct((B, N, C0), _F32),
            jax.ShapeDtypeStruct((2, C0), _F32),
        ],
    )(*inputs)
    return x, st


def _mlp_body(x_ref, sp_ref, g_ref, be_ref, wt_ref, b_ref, o_ref, *rest,
              cnt, out_stats):
    sc, sh = _finalize_stats(sp_ref, g_ref, be_ref, cnt)
    a = jnp.maximum(x_ref[...] * sc[None, :] + sh[None, :], 0.0)
    y = jnp.dot(a, wt_ref[...], preferred_element_type=_F32) + b_ref[0, :][None, :]
    o_ref[...] = y
    if out_stats:
        st_ref = rest[0]
        _accum_stats(st_ref, y, pl.program_id(0) == 0)


def _mlp_layer(x, st_prev, g_prev, be_prev, w, b, rt, out_stats=True):
    R, Cp = x.shape
    Co = w.shape[0]
    wt = jnp.transpose(w)
    cnt = float(R)

    inputs = [x, st_prev, g_prev.reshape(1, -1), be_prev.reshape(1, -1),
              wt, b.reshape(1, -1)]
    specs = [
        pl.BlockSpec((rt, Cp), lambda i: (i, 0)),
        pl.BlockSpec((2, Cp), lambda i: (0, 0)),
        pl.BlockSpec((1, Cp), lambda i: (0, 0)),
        pl.BlockSpec((1, Cp), lambda i: (0, 0)),
        pl.BlockSpec((Cp, Co), lambda i: (0, 0)),
        pl.BlockSpec((1, Co), lambda i: (0, 0)),
    ]
    out_specs = [pl.BlockSpec((rt, Co), lambda i: (i, 0))]
    out_shape = [jax.ShapeDtypeStruct((R, Co), _F32)]
    if out_stats:
        out_specs.append(pl.BlockSpec((2, Co), lambda i: (0, 0)))
        out_shape.append(jax.ShapeDtypeStruct((2, Co), _F32))

    body = functools.partial(_mlp_body, cnt=cnt, out_stats=out_stats)
    res = pl.pallas_call(
        body,
        grid=(R // rt,),
        in_specs=specs,
        out_specs=out_specs,
        out_shape=out_shape,
    )(*inputs)
    if out_stats:
        return res[0], res[1]
    return res[0]


def kernel(xyz0, xyz1, xyz2, xyz3, xyz4, pts1, pts2, pts3, pts4, params):
    p = params
    B = xyz0.shape[0]

    x, st = _level_pass_a(xyz3, xyz4, pts4, None, None, None, pts3,
                          p['fp4_w0'], p['fp4_b0'], qt=64)
    x = x.reshape(B * 64, -1)
    x, st = _mlp_layer(x, st, p['fp4_g0'], p['fp4_be0'],
                       p['fp4_w1'], p['fp4_b1'], rt=B * 64)

    x, st = _level_pass_a(xyz2, xyz3, x.reshape(B, 64, -1), st,
                          p['fp4_g1'], p['fp4_be1'], pts2,
                          p['fp3_w0'], p['fp3_b0'], qt=256)
    x = x.reshape(B * 256, -1)
    x, st = _mlp_layer(x, st, p['fp3_g0'], p['fp3_be0'],
                       p['fp3_w1'], p['fp3_b1'], rt=B * 256)

    x, st = _level_pass_a(xyz1, xyz2, x.reshape(B, 256, -1), st,
                          p['fp3_g1'], p['fp3_be1'], pts1,
                          p['fp2_w0'], p['fp2_b0'], qt=512)
    x = x.reshape(B * 1024, -1)
    x, st = _mlp_layer(x, st, p['fp2_g0'], p['fp2_be0'],
                       p['fp2_w1'], p['fp2_b1'], rt=2048)

    x, st = _level_pass_a(xyz0, xyz1, x.reshape(B, 1024, -1), st,
                          p['fp2_g1'], p['fp2_be1'], None,
                          p['fp1_w0'], p['fp1_b0'], qt=1024)
    x = x.reshape(B * 4096, -1)
    x, st = _mlp_layer(x, st, p['fp1_g0'], p['fp1_be0'],
                       p['fp1_w1'], p['fp1_b1'], rt=4096)
    x, st = _mlp_layer(x, st, p['fp1_g1'], p['fp1_be1'],
                       p['fp1_w2'], p['fp1_b2'], rt=4096)

    x, st = _mlp_layer(x, st, p['fp1_g2'], p['fp1_be2'],
                       p['conv1_w'], p['conv1_b'], rt=4096)
    x = _mlp_layer(x, st, p['bn1_g'], p['bn1_be'],
                   p['conv2_w'], p['conv2_b'], rt=4096, out_stats=False)
    return x.reshape(B, 4096, -1)


# speedup vs baseline: 13.2594x; 13.2594x over previous
import functools

import jax
import jax.numpy as jnp
from jax.experimental import pallas as pl

_F32 = jnp.float32
_BN_EPS = 1e-5
_BIG = 1e30


def _mm_bf16(a, b):
    return jnp.dot(a.astype(jnp.bfloat16), b.astype(jnp.bfloat16),
                   preferred_element_type=_F32)


def _mm_bf16x3(a, b):
    ah = a.astype(jnp.bfloat16)
    al = (a - ah.astype(_F32)).astype(jnp.bfloat16)
    bh = b.astype(jnp.bfloat16)
    bl = (b - bh.astype(_F32)).astype(jnp.bfloat16)
    d = lambda u, v: jnp.dot(u, v, preferred_element_type=_F32)
    return (d(al, bh) + d(ah, bl)) + d(ah, bh)


def _finalize_stats(st_ref, g_ref, be_ref, cnt):
    s = st_ref[0, :]
    ss = st_ref[1, :]
    mu = s / cnt
    var = ss / cnt - mu * mu
    sc = g_ref[0, :] / jnp.sqrt(var + _BN_EPS)
    sh = be_ref[0, :] - mu * sc
    return sc, sh


def _accum_stats(st_ref, x, first):
    ps = jnp.sum(x, axis=0)
    pss = jnp.sum(x * x, axis=0)
    part = jnp.concatenate([ps[None, :], pss[None, :]], axis=0)
    prev = st_ref[...]
    st_ref[...] = jnp.where(first, part, prev + part)


def _interp_body(refs, *, M, cnt_prev, has_p1, norm_p2, dist_bf16,
                 emit_interp=False):
    i = 0
    xyz1_ref = refs[i]; i += 1
    xyz2_ref = refs[i]; i += 1
    p2_ref = refs[i]; i += 1
    if norm_p2:
        sp_ref = refs[i]; i += 1
        gp_ref = refs[i]; i += 1
        bep_ref = refs[i]; i += 1
    if has_p1:
        p1_ref = refs[i]; i += 1
        w0a_ref = refs[i]; i += 1
    w0b_ref = refs[i]; i += 1
    b0_ref = refs[i]; i += 1
    x_ref = refs[i]; i += 1
    st_ref = refs[i]; i += 1

    b = pl.program_id(0)
    t = pl.program_id(1)
    first = jnp.logical_and(b == 0, t == 0)

    q = xyz1_ref[0]
    k = xyz2_ref[0]
    p2 = p2_ref[0]
    if norm_p2:
        sc, sh = _finalize_stats(sp_ref, gp_ref, bep_ref, cnt_prev)
        p2 = jnp.maximum(p2 * sc[None, :] + sh[None, :], 0.0)

    qq = jnp.sum(q * q, axis=1, keepdims=True)
    kk = jnp.sum(k * k, axis=1)[None, :]
    if dist_bf16:
        qk = jax.lax.dot_general(q.astype(jnp.bfloat16),
                                 k.astype(jnp.bfloat16),
                                 (((1,), (1,)), ((), ())),
                                 preferred_element_type=_F32)
    else:
        qk = jax.lax.dot_general(q, k, (((1,), (1,)), ((), ())),
                                 preferred_element_type=_F32,
                                 precision=jax.lax.Precision.HIGHEST)
    d = (-2.0 * qk + qq) + kk

    iota = jax.lax.broadcasted_iota(jnp.int32, d.shape, 1)
    dd = d
    recs = []
    sels = []
    for _ in range(3):
        mv = jnp.min(dd, axis=1, keepdims=True)
        idx = jnp.min(jnp.where(dd == mv, iota, M), axis=1, keepdims=True)
        sel = iota == idx
        dd = jnp.where(sel, _BIG, dd)
        recs.append(1.0 / (mv + 1e-8))
        sels.append(sel)
    wsum = recs[0] + recs[1] + recs[2]

    fs = [jnp.dot(sels[j].astype(_F32), p2, preferred_element_type=_F32,
                  precision=jax.lax.Precision.HIGHEST) for j in range(3)]
    interp = ((recs[0] / wsum) * fs[0]
              + (recs[1] / wsum) * fs[1]
              + (recs[2] / wsum) * fs[2])
    if emit_interp:
        x_ref[0] = interp
        _accum_stats(st_ref, interp, first)
        return
    x = _mm_bf16(interp, w0b_ref[...])
    if has_p1:
        x = x + _mm_bf16(p1_ref[0], w0a_ref[...])
    x = x + b0_ref[0, :][None, :]
    x_ref[0] = x
    _accum_stats(st_ref, x, first)


def _level_pass_a(xyz1, xyz2, p2, st_prev, g_prev, be_prev, p1, w0, b0, qt,
                  dist_bf16=True, emit_interp=False):
    B, N, _ = xyz1.shape
    M = xyz2.shape[1]
    C2 = p2.shape[2]
    C0 = C2 if emit_interp else w0.shape[0]
    norm_p2 = st_prev is not None
    has_p1 = p1 is not None
    cnt_prev = float(B * M)

    inputs = [xyz1, xyz2, p2]
    specs = [
        pl.BlockSpec((1, qt, 3), lambda b, t: (b, t, 0)),
        pl.BlockSpec((1, M, 3), lambda b, t: (b, 0, 0)),
        pl.BlockSpec((1, M, C2), lambda b, t: (b, 0, 0)),
    ]
    if norm_p2:
        inputs += [st_prev, g_prev.reshape(1, -1), be_prev.reshape(1, -1)]
        specs += [
            pl.BlockSpec((2, C2), lambda b, t: (0, 0)),
            pl.BlockSpec((1, C2), lambda b, t: (0, 0)),
            pl.BlockSpec((1, C2), lambda b, t: (0, 0)),
        ]
    if has_p1:
        C1 = p1.shape[2]
        w0a = jnp.transpose(w0[:, :C1])
        w0b = jnp.transpose(w0[:, C1:])
        inputs += [p1, w0a]
        specs += [
            pl.BlockSpec((1, qt, C1), lambda b, t: (b, t, 0)),
            pl.BlockSpec((C1, C0), lambda b, t: (0, 0)),
        ]
    else:
        w0b = jnp.transpose(w0)
    inputs += [w0b, b0.reshape(1, -1)]
    specs += [
        pl.BlockSpec((C2, C0), lambda b, t: (0, 0)),
        pl.BlockSpec((1, C0), lambda b, t: (0, 0)),
    ]

    body = functools.partial(
        _interp_body, M=M, cnt_prev=cnt_prev, has_p1=has_p1, norm_p2=norm_p2,
        dist_bf16=dist_bf16, emit_interp=emit_interp)

    def wrapped(*refs):
        body(refs)

    x, st = pl.pallas_call(
        wrapped,
        grid=(B, N // qt),
        in_specs=specs,
        out_specs=[
            pl.BlockSpec((1, qt, C0), lambda b, t: (b, t, 0)),
            pl.BlockSpec((2, C0), lambda b, t: (0, 0)),
        ],
        out_shape=[
            jax.ShapeDtypeStruct((B, N, C0), _F32),
            jax.ShapeDtypeStruct((2, C0), _F32),
        ],
    )(*inputs)
    return x, st


def _mlp_body(x_ref, sp_ref, g_ref, be_ref, wt_ref, b_ref, o_ref, *rest,
              cnt, out_stats):
    sc, sh = _finalize_stats(sp_ref, g_ref, be_ref, cnt)
    a = jnp.maximum(x_ref[...] * sc[None, :] + sh[None, :], 0.0)
    y = _mm_bf16(a, wt_ref[...]) + b_ref[0, :][None, :]
    o_ref[...] = y
    if out_stats:
        st_ref = rest[0]
        _accum_stats(st_ref, y, pl.program_id(0) == 0)


def _mlp_layer(x, st_prev, g_prev, be_prev, w, b, rt, out_stats=True):
    R, Cp = x.shape
    Co = w.shape[0]
    wt = jnp.transpose(w)
    cnt = float(R)

    inputs = [x, st_prev, g_prev.reshape(1, -1), be_prev.reshape(1, -1),
              wt, b.reshape(1, -1)]
    specs = [
        pl.BlockSpec((rt, Cp), lambda i: (i, 0)),
        pl.BlockSpec((2, Cp), lambda i: (0, 0)),
        pl.BlockSpec((1, Cp), lambda i: (0, 0)),
        pl.BlockSpec((1, Cp), lambda i: (0, 0)),
        pl.BlockSpec((Cp, Co), lambda i: (0, 0)),
        pl.BlockSpec((1, Co), lambda i: (0, 0)),
    ]
    out_specs = [pl.BlockSpec((rt, Co), lambda i: (i, 0))]
    out_shape = [jax.ShapeDtypeStruct((R, Co), _F32)]
    if out_stats:
        out_specs.append(pl.BlockSpec((2, Co), lambda i: (0, 0)))
        out_shape.append(jax.ShapeDtypeStruct((2, Co), _F32))

    body = functools.partial(_mlp_body, cnt=cnt, out_stats=out_stats)
    res = pl.pallas_call(
        body,
        grid=(R // rt,),
        in_specs=specs,
        out_specs=out_specs,
        out_shape=out_shape,
    )(*inputs)
    if out_stats:
        return res[0], res[1]
    return res[0]


def kernel(xyz0, xyz1, xyz2, xyz3, xyz4, pts1, pts2, pts3, pts4, params):
    if _DIAG:
        return _diag_kernel(xyz0, xyz1, xyz2, xyz3, xyz4, pts1, pts2, pts3, pts4, params)
    p = params
    B = xyz0.shape[0]

    x, st = _level_pass_a(xyz3, xyz4, pts4, None, None, None, pts3,
                          p['fp4_w0'], p['fp4_b0'], qt=64)
    x = x.reshape(B * 64, -1)
    x, st = _mlp_layer(x, st, p['fp4_g0'], p['fp4_be0'],
                       p['fp4_w1'], p['fp4_b1'], rt=B * 64)

    x, st = _level_pass_a(xyz2, xyz3, x.reshape(B, 64, -1), st,
                          p['fp4_g1'], p['fp4_be1'], pts2,
                          p['fp3_w0'], p['fp3_b0'], qt=256)
    x = x.reshape(B * 256, -1)
    x, st = _mlp_layer(x, st, p['fp3_g0'], p['fp3_be0'],
                       p['fp3_w1'], p['fp3_b1'], rt=B * 256)

    x, st = _level_pass_a(xyz1, xyz2, x.reshape(B, 256, -1), st,
                          p['fp3_g1'], p['fp3_be1'], pts1,
                          p['fp2_w0'], p['fp2_b0'], qt=512)
    x = x.reshape(B * 1024, -1)
    x, st = _mlp_layer(x, st, p['fp2_g0'], p['fp2_be0'],
                       p['fp2_w1'], p['fp2_b1'], rt=2048)

    x, st = _level_pass_a(xyz0, xyz1, x.reshape(B, 1024, -1), st,
                          p['fp2_g1'], p['fp2_be1'], None,
                          p['fp1_w0'], p['fp1_b0'], qt=1024)
    x = x.reshape(B * 4096, -1)
    x, st = _mlp_layer(x, st, p['fp1_g0'], p['fp1_be0'],
                       p['fp1_w1'], p['fp1_b1'], rt=4096)
    x, st = _mlp_layer(x, st, p['fp1_g1'], p['fp1_be1'],
                       p['fp1_w2'], p['fp1_b2'], rt=4096)

    x, st = _mlp_layer(x, st, p['fp1_g2'], p['fp1_be2'],
                       p['conv1_w'], p['conv1_b'], rt=4096)
    x = _mlp_layer(x, st, p['bn1_g'], p['bn1_be'],
                   p['conv2_w'], p['conv2_b'], rt=4096, out_stats=False)
    return x.reshape(B, 4096, -1)


_DIAG = 0


def _ref_fp_layers(name, xyz1, xyz2, points1, points2, params):
    d = (-2.0 * jnp.einsum('bnd,bmd->bnm', xyz1, xyz2)
         + jnp.sum(xyz1 * xyz1, axis=-1)[:, :, None]
         + jnp.sum(xyz2 * xyz2, axis=-1)[:, None, :])
    idx = jnp.argsort(d, axis=-1)[:, :, :3]
    dists = jnp.take_along_axis(d, idx, axis=-1)
    recip = 1.0 / (dists + 1e-8)
    weight = recip / jnp.sum(recip, axis=2, keepdims=True)
    gathered = jnp.take_along_axis(points2[:, None, :, :], idx[:, :, :, None], axis=2)
    new = jnp.sum(gathered * weight[..., None], axis=2)
    if points1 is not None:
        new = jnp.concatenate([points1, new], axis=-1)
    outs = []
    j = 0
    while ('%s_w%d' % (name, j)) in params:
        w = params['%s_w%d' % (name, j)]
        b = params['%s_b%d' % (name, j)]
        g = params['%s_g%d' % (name, j)]
        be = params['%s_be%d' % (name, j)]
        new = new @ w.T + b
        mu = jnp.mean(new, axis=(0, 1))
        var = jnp.var(new, axis=(0, 1))
        new = (new - mu) / jnp.sqrt(var + 1e-5) * g + be
        new = jax.nn.relu(new)
        outs.append(new)
        j += 1
    return outs


def _norm_jnp(x, st, g, be, cnt):
    mu = st[0] / cnt
    var = st[1] / cnt - mu * mu
    sc = g / jnp.sqrt(var + _BN_EPS)
    sh = be - mu * sc
    return jnp.maximum(x * sc + sh, 0.0)


def _diag_kernel(xyz0, xyz1, xyz2, xyz3, xyz4, pts1, pts2, pts3, pts4, params):
    p = params
    B = 8
    r0, r1 = _ref_fp_layers('fp4', xyz3, xyz4, pts3, pts4, p)
    x, st = _level_pass_a(xyz3, xyz4, pts4, None, None, None, pts3,
                          p['fp4_w0'], p['fp4_b0'], qt=64)
    x4a = x
    m0 = _norm_jnp(x.reshape(B * 64, -1), st, p['fp4_g0'], p['fp4_be0'], 512.0)
    x2, st2 = _mlp_layer(x.reshape(B * 64, -1), st, p['fp4_g0'], p['fp4_be0'],
                         p['fp4_w1'], p['fp4_b1'], rt=B * 64)
    m1 = _norm_jnp(x2, st2, p['fp4_g1'], p['fp4_be1'], 512.0)
    r3 = _ref_fp_layers('fp3', xyz2, xyz3, pts2, r1, p)[-1]
    x, st = _level_pass_a(xyz2, xyz3, x2.reshape(B, 64, -1), st2,
                          p['fp4_g1'], p['fp4_be1'], pts2,
                          p['fp3_w0'], p['fp3_b0'], qt=256)
    x, st = _mlp_layer(x.reshape(B * 256, -1), st, p['fp3_g0'], p['fp3_be0'],
                       p['fp3_w1'], p['fp3_b1'], rt=B * 256)
    m3 = _norm_jnp(x, st, p['fp3_g1'], p['fp3_be1'], 2048.0)
    c3 = jnp.sum(jnp.abs(m3 - r3.reshape(B * 256, -1)) > 0.01)
    r2 = _ref_fp_layers('fp2', xyz1, xyz2, pts1, r3, p)[-1]
    x, st = _level_pass_a(xyz1, xyz2, x.reshape(B, 256, -1), st,
                          p['fp3_g1'], p['fp3_be1'], pts1,
                          p['fp2_w0'], p['fp2_b0'], qt=512)
    x, st = _mlp_layer(x.reshape(B * 1024, -1), st, p['fp2_g0'], p['fp2_be0'],
                       p['fp2_w1'], p['fp2_b1'], rt=2048)
    m2 = _norm_jnp(x, st, p['fp2_g1'], p['fp2_be1'], 8192.0)
    c2 = jnp.sum(jnp.abs(m2 - r2.reshape(B * 1024, -1)) > 0.01)
    r1f = _ref_fp_layers('fp1', xyz0, xyz1, None, r2, p)[-1]
    x, st = _level_pass_a(xyz0, xyz1, x.reshape(B, 1024, -1), st,
                          p['fp2_g1'], p['fp2_be1'], None,
                          p['fp1_w0'], p['fp1_b0'], qt=1024)
    x, st = _mlp_layer(x.reshape(B * 4096, -1), st, p['fp1_g0'], p['fp1_be0'],
                       p['fp1_w1'], p['fp1_b1'], rt=4096)
    x, st = _mlp_layer(x, st, p['fp1_g1'], p['fp1_be1'],
                       p['fp1_w2'], p['fp1_b2'], rt=4096)
    m1f = _norm_jnp(x, st, p['fp1_g2'], p['fp1_be2'], 32768.0)
    c1f = jnp.sum(jnp.abs(m1f - r1f.reshape(B * 4096, -1)) > 0.01)
    dref = (-2.0 * jnp.einsum('bnd,bmd->bnm', xyz3, xyz4)
            + jnp.sum(xyz3 * xyz3, axis=-1)[:, :, None]
            + jnp.sum(xyz4 * xyz4, axis=-1)[:, None, :])
    idxr = jnp.argsort(dref, axis=-1)[:, :, :3]
    dsr = jnp.take_along_axis(dref, idxr, axis=-1)
    rcr = 1.0 / (dsr + 1e-8)
    wr = rcr / jnp.sum(rcr, axis=2, keepdims=True)
    gr = jnp.take_along_axis(pts4[:, None, :, :], idxr[:, :, :, None], axis=2)
    ref_interp = jnp.sum(gr * wr[..., None], axis=2)
    my_interp, _ = _level_pass_a(xyz3, xyz4, pts4, None, None, None, pts3,
                                 p['fp4_w0'], p['fp4_b0'], qt=64,
                                 emit_interp=True)
    cat = jnp.concatenate([pts3, ref_interp], axis=-1).reshape(B * 64, -1)
    wt4 = p['fp4_w0'].T
    y_def = cat @ wt4
    y_hi = jnp.dot(cat, wt4, precision=jax.lax.Precision.HIGHEST)
    ah = cat.astype(jnp.bfloat16)
    al = (cat - ah.astype(jnp.float32)).astype(jnp.bfloat16)
    bh = wt4.astype(jnp.bfloat16)
    bl = (wt4 - bh.astype(jnp.float32)).astype(jnp.bfloat16)
    dd = lambda u, v: jnp.dot(u, v, preferred_element_type=jnp.float32)
    y_3x = (dd(al, bh) + dd(ah, bl)) + dd(ah, bh)
    mxp = jnp.max(jnp.abs(x4a.reshape(B * 64, -1) - y_def - p['fp4_b0']))
    mx0 = jnp.max(jnp.abs(m0 - r0.reshape(B * 64, -1)))
    beacon = (jnp.floor(jnp.minimum(mxp * 1e7, 9999.0)) * 1000.0
              + jnp.floor(jnp.minimum(mx0 * 1e7, 999.0)))
    out = jnp.zeros((B, 4096, 128), jnp.float32)
    return out.at[0, 0, 0].set(beacon * 1000.0)
